# baseline (speedup 1.0000x reference)
import functools

import jax
import jax.numpy as jnp
from jax import lax
from jax.experimental import pallas as pl
from jax.experimental.pallas import tpu as pltpu
from jax.experimental.pallas import tpu_sc as plsc

N_NODES = 10000
N_EDGES = 160000
H = 32
W_AUG = 48
NC, NS, L = 2, 16, 16
NW = NC * NS
EB = 128
EPT = 5120
E_PAD = NW * EPT
NBLK_FULL = EPT // EB
NBLK_LAST = (N_EDGES - (NW - 1) * EPT) // EB
N_PAD = 10240
ROWS_PER_SUB = N_PAD // NS
R_TC = 1000
GRID_TC = N_NODES // R_TC


def _sc_mesh():
    return plsc.VectorSubcoreMesh(
        core_axis_name="c", subcore_axis_name="s",
        num_cores=NC, num_subcores=NS)


def _vmem_to_vmem(dst, dst_off, src, src_off, n):
    for k in range(n // L):
        dst[pl.ds(dst_off + k * L, L)] = src[pl.ds(src_off + k * L, L)]


@functools.cache
def _build_edge_phase():
    @functools.partial(
        pl.kernel,
        out_type=jax.ShapeDtypeStruct((NC, N_PAD, W_AUG), jnp.float32),
        mesh=_sc_mesh(),
        compiler_params=pltpu.CompilerParams(use_tc_tiling_on_sc=False),
        scratch_types=[
            pltpu.VMEM((EPT,), jnp.int32),
            pltpu.VMEM((EPT,), jnp.int32),
            pltpu.VMEM((EB,), jnp.int32),
            pltpu.VMEM((EB,), jnp.int32),
            pltpu.VMEM((EB, H), jnp.float32),
            pltpu.VMEM((EB, H), jnp.float32),
            pltpu.VMEM((EB, H), jnp.float32),
            pltpu.VMEM((EB, H), jnp.float32),
            pltpu.VMEM((EB, W_AUG), jnp.float32),
            pltpu.VMEM((EB, W_AUG), jnp.float32),
            pltpu.VMEM((H,), jnp.float32),
            pltpu.VMEM((ROWS_PER_SUB, W_AUG), jnp.float32),
            pltpu.VMEM_SHARED((N_PAD, W_AUG), jnp.float32),
            pltpu.SemaphoreType.DMA,
            pltpu.SemaphoreType.DMA,
            pltpu.SemaphoreType.DMA,
            pltpu.SemaphoreType.DMA,
        ],
    )
    def _edge_phase(hl_hbm, hr_hbm, src_hbm, dst_hbm, att_hbm, aug_out,
                    sidx_all, didx_all, didx_a, didx_b,
                    hl_a, hl_b, hr_a, hr_b, pv_a, pv_b,
                    att_v, zero_v, acc_sh,
                    sem_ga, sem_gb, sem_sa, sem_sb):
        c = lax.axis_index("c")
        s = lax.axis_index("s")
        wid = s * NC + c
        base_e = wid * EPT
        nblk = jnp.where(wid == NW - 1, NBLK_LAST, NBLK_FULL)

        lane_i = lax.broadcasted_iota(jnp.int32, (L,), 0)
        zv = lane_i.astype(jnp.float32) * 0.0

        pltpu.sync_copy(src_hbm.at[pl.ds(base_e, EPT)], sidx_all)
        pltpu.sync_copy(dst_hbm.at[pl.ds(base_e, EPT)], didx_all)
        pltpu.sync_copy(att_hbm, att_v)
        att0 = att_v[pl.ds(0, L)]
        att1 = att_v[pl.ds(L, L)]
        onehot0 = jnp.where(lane_i == 0, 1.0, 0.0).astype(jnp.float32)

        def _zrow(i, carry):
            for k in range(W_AUG // L):
                zero_v[i, pl.ds(k * L, L)] = zv
            return carry

        lax.fori_loop(0, ROWS_PER_SUB, _zrow, 0)
        pltpu.sync_copy(zero_v,
                        acc_sh.at[pl.ds(s * ROWS_PER_SUB, ROWS_PER_SUB)])

        def _gissue(j, hl_p, hr_p, sem_g):
            pltpu.async_copy(
                hl_hbm.at[sidx_all.at[pl.ds(j * EB, EB)]], hl_p, sem_g)
            pltpu.async_copy(
                hr_hbm.at[didx_all.at[pl.ds(j * EB, EB)]], hr_p, sem_g)

        def _gwait(hl_p, hr_p, sem_g):
            pltpu.make_async_copy(hl_hbm.at[pl.ds(0, EB)], hl_p, sem_g).wait()
            pltpu.make_async_copy(hr_hbm.at[pl.ds(0, EB)], hr_p, sem_g).wait()

        def _compute(hl_p, hr_p, pv_p):
            @plsc.parallel_loop(0, EB, unroll=8)
            def _edge(e):
                hl0 = hl_p[e, pl.ds(0, L)]
                hl1 = hl_p[e, pl.ds(L, L)]
                hr0 = hr_p[e, pl.ds(0, L)]
                hr1 = hr_p[e, pl.ds(L, L)]
                s0 = hl0 + hr0
                s1 = hl1 + hr1
                t0 = jnp.maximum(s0, 0.2 * s0)
                t1 = jnp.maximum(s1, 0.2 * s1)
                u = t0 * att0 + t1 * att1
                for sh in (8, 4, 2, 1):
                    u = u + u.at[lane_i ^ sh].get(mode="promise_in_bounds")
                p = jnp.exp(u)
                pv_p[e, pl.ds(0, L)] = p * hl0
                pv_p[e, pl.ds(L, L)] = p * hl1
                pv_p[e, pl.ds(2 * L, L)] = p * onehot0

        def _swait(pv_p, didx_p, sem_s):
            pltpu.make_async_copy(pv_p, acc_sh.at[didx_p], sem_s).wait()

        plsc.subcore_barrier()

        _gissue(0, hl_a, hr_a, sem_ga)

        def _half(j, hl_p, hr_p, pv_p, didx_p, sem_g, sem_s,
                  hl_q, hr_q, sem_gq):
            @pl.when(j < nblk)
            def _():
                _gwait(hl_p, hr_p, sem_g)

                @pl.when(j + 1 < nblk)
                def _():
                    _gissue(j + 1, hl_q, hr_q, sem_gq)

                @pl.when(j >= 2)
                def _():
                    _swait(pv_p, didx_p, sem_s)

                _compute(hl_p, hr_p, pv_p)
                _vmem_to_vmem(didx_p, 0, didx_all, j * EB, EB)
                pltpu.async_copy(pv_p, acc_sh.at[didx_p], sem_s, add=True)

        def _pair(jj, carry):
            j0 = 2 * jj
            _half(j0, hl_a, hr_a, pv_a, didx_a, sem_ga, sem_sa,
                  hl_b, hr_b, sem_gb)
            _half(j0 + 1, hl_b, hr_b, pv_b, didx_b, sem_gb, sem_sb,
                  hl_a, hr_a, sem_ga)
            return carry

        lax.fori_loop(0, NBLK_FULL // 2, _pair, 0)

        _swait(pv_a, didx_a, sem_sa)
        _swait(pv_b, didx_b, sem_sb)

        plsc.subcore_barrier()
        pltpu.sync_copy(acc_sh.at[pl.ds(s * ROWS_PER_SUB, ROWS_PER_SUB)],
                        aug_out.at[c, pl.ds(s * ROWS_PER_SUB, ROWS_PER_SUB)])

    return _edge_phase


@functools.cache
def _build_decoder():
    @functools.partial(
        pl.kernel,
        out_type=jax.ShapeDtypeStruct((E_PAD,), jnp.float32),
        mesh=_sc_mesh(),
        compiler_params=pltpu.CompilerParams(use_tc_tiling_on_sc=False),
        scratch_types=[
            pltpu.VMEM((EPT,), jnp.int32),
            pltpu.VMEM((EPT,), jnp.int32),
            pltpu.VMEM((EB, H), jnp.float32),
            pltpu.VMEM((EB, H), jnp.float32),
            pltpu.VMEM((EB, H), jnp.float32),
            pltpu.VMEM((EB, H), jnp.float32),
            pltpu.VMEM((EPT,), jnp.float32),
            pltpu.VMEM((H,), jnp.float32),
            pltpu.VMEM((L,), jnp.float32),
            pltpu.SemaphoreType.DMA,
            pltpu.SemaphoreType.DMA,
        ],
    )
    def _decoder(p_hbm, q_hbm, row_hbm, col_hbm, w2_hbm, b2_hbm, pred_out,
                 ridx_all, cidx_all, pr_a, pr_b, qr_a, qr_b,
                 pred_v, w2_v, b2_v, sem_ga, sem_gb):
        c = lax.axis_index("c")
        s = lax.axis_index("s")
        wid = s * NC + c
        base_e = wid * EPT
        nblk = jnp.where(wid == NW - 1, NBLK_LAST, NBLK_FULL)

        pltpu.sync_copy(row_hbm.at[pl.ds(base_e, EPT)], ridx_all)
        pltpu.sync_copy(col_hbm.at[pl.ds(base_e, EPT)], cidx_all)
        pltpu.sync_copy(w2_hbm, w2_v)
        pltpu.sync_copy(b2_hbm, b2_v)
        w20 = w2_v[pl.ds(0, L)]
        w21 = w2_v[pl.ds(L, L)]
        b2 = b2_v[pl.ds(0, L)]
        lane_i = lax.broadcasted_iota(jnp.int32, (L,), 0)
        zv = lane_i.astype(jnp.float32) * 0.0

        def _gissue(j, pr_p, qr_p, sem_g):
            pltpu.async_copy(
                p_hbm.at[ridx_all.at[pl.ds(j * EB, EB)]], pr_p, sem_g)
            pltpu.async_copy(
                q_hbm.at[cidx_all.at[pl.ds(j * EB, EB)]], qr_p, sem_g)

        def _gwait(pr_p, qr_p, sem_g):
            pltpu.make_async_copy(p_hbm.at[pl.ds(0, EB)], pr_p, sem_g).wait()
            pltpu.make_async_copy(q_hbm.at[pl.ds(0, EB)], qr_p, sem_g).wait()

        def _compute(j, pr_p, qr_p):
            @plsc.parallel_loop(0, EB // L, unroll=4)
            def _grp(g):
                base = g * L
                pacc = zv
                for jj in range(L):
                    e = base + jj
                    p0 = pr_p[e, pl.ds(0, L)]
                    p1 = pr_p[e, pl.ds(L, L)]
                    q0 = qr_p[e, pl.ds(0, L)]
                    q1 = qr_p[e, pl.ds(L, L)]
                    t0 = jnp.maximum(p0 + q0, 0.0)
                    t1 = jnp.maximum(p1 + q1, 0.0)
                    u = t0 * w20 + t1 * w21
                    for sh in (8, 4, 2, 1):
                        u = u + u.at[lane_i ^ sh].get(
                            mode="promise_in_bounds")
                    pacc = jnp.where(lane_i == jj, u, pacc)
                pred_v[pl.ds(j * EB + base, L)] = pacc + b2

        _gissue(0, pr_a, qr_a, sem_ga)

        def _half(j, pr_p, qr_p, sem_g, pr_q, qr_q, sem_gq):
            @pl.when(j < nblk)
            def _():
                _gwait(pr_p, qr_p, sem_g)

                @pl.when(j + 1 < nblk)
                def _():
                    _gissue(j + 1, pr_q, qr_q, sem_gq)

                _compute(j, pr_p, qr_p)

        def _pair(jj, carry):
            j0 = 2 * jj
            _half(j0, pr_a, qr_a, sem_ga, pr_b, qr_b, sem_gb)
            _half(j0 + 1, pr_b, qr_b, sem_gb, pr_a, qr_a, sem_ga)
            return carry

        lax.fori_loop(0, NBLK_FULL // 2, _pair, 0)

        pltpu.sync_copy(pred_v, pred_out.at[pl.ds(base_e, EPT)])

    return _decoder


def _proj1_body(xu, xi, wlr, blr, wrr, brr, wlv, blv, wrv, brv,
                hlr, hrr, hlv, hrv):
    xu_ = xu[...]
    xi_ = xi[...]
    f32 = jnp.float32
    hlr[...] = jnp.dot(xu_, wlr[...], preferred_element_type=f32, precision=lax.Precision.HIGHEST) + blr[...]
    hrr[...] = jnp.dot(xi_, wrr[...], preferred_element_type=f32, precision=lax.Precision.HIGHEST) + brr[...]
    hlv[...] = jnp.dot(xi_, wlv[...], preferred_element_type=f32, precision=lax.Precision.HIGHEST) + blv[...]
    hrv[...] = jnp.dot(xu_, wrv[...], preferred_element_type=f32, precision=lax.Precision.HIGHEST) + brv[...]


_proj1 = pl.pallas_call(
    _proj1_body,
    grid=(GRID_TC,),
    in_specs=[
        pl.BlockSpec((R_TC, 128), lambda i: (i, 0)),
        pl.BlockSpec((R_TC, 256), lambda i: (i, 0)),
        pl.BlockSpec((128, H), lambda i: (0, 0)),
        pl.BlockSpec((1, H), lambda i: (0, 0)),
        pl.BlockSpec((256, H), lambda i: (0, 0)),
        pl.BlockSpec((1, H), lambda i: (0, 0)),
        pl.BlockSpec((256, H), lambda i: (0, 0)),
        pl.BlockSpec((1, H), lambda i: (0, 0)),
        pl.BlockSpec((128, H), lambda i: (0, 0)),
        pl.BlockSpec((1, H), lambda i: (0, 0)),
    ],
    out_specs=[pl.BlockSpec((R_TC, H), lambda i: (i, 0))] * 4,
    out_shape=[jax.ShapeDtypeStruct((N_NODES, H), jnp.float32)] * 4,
)


def _mid_body(augr, augv, b1r, b1v, wl2r, bl2r, wr2r, br2r,
              wl2v, bl2v, wr2v, br2v, hl2r, hr2r, hl2v, hr2v):
    f32 = jnp.float32
    ar = augr[0] + augr[1]
    av = augv[0] + augv[1]
    zi1 = jnp.maximum(ar[:, :H] / (ar[:, H:H + 1] + 1e-16) + b1r[...], 0.0)
    zu1 = jnp.maximum(av[:, :H] / (av[:, H:H + 1] + 1e-16) + b1v[...], 0.0)
    hl2r[...] = jnp.dot(zu1, wl2r[...], preferred_element_type=f32, precision=lax.Precision.HIGHEST) + bl2r[...]
    hr2r[...] = jnp.dot(zi1, wr2r[...], preferred_element_type=f32, precision=lax.Precision.HIGHEST) + br2r[...]
    hl2v[...] = jnp.dot(zi1, wl2v[...], preferred_element_type=f32, precision=lax.Precision.HIGHEST) + bl2v[...]
    hr2v[...] = jnp.dot(zu1, wr2v[...], preferred_element_type=f32, precision=lax.Precision.HIGHEST) + br2v[...]


_mid = pl.pallas_call(
    _mid_body,
    grid=(GRID_TC,),
    in_specs=[
        pl.BlockSpec((NC, R_TC, W_AUG), lambda i: (0, i, 0)),
        pl.BlockSpec((NC, R_TC, W_AUG), lambda i: (0, i, 0)),
        pl.BlockSpec((1, H), lambda i: (0, 0)),
        pl.BlockSpec((1, H), lambda i: (0, 0)),
    ] + [
        pl.BlockSpec((H, H), lambda i: (0, 0)),
        pl.BlockSpec((1, H), lambda i: (0, 0)),
    ] * 4,
    out_specs=[pl.BlockSpec((R_TC, H), lambda i: (i, 0))] * 4,
    out_shape=[jax.ShapeDtypeStruct((N_NODES, H), jnp.float32)] * 4,
)


def _fin_body(augr, augv, b2r, b2v, w1u, w1i, b1d, p_out, q_out):
    f32 = jnp.float32
    ar = augr[0] + augr[1]
    av = augv[0] + augv[1]
    zi2 = ar[:, :H] / (ar[:, H:H + 1] + 1e-16) + b2r[...]
    zu2 = av[:, :H] / (av[:, H:H + 1] + 1e-16) + b2v[...]
    p_out[...] = jnp.dot(zu2, w1u[...], preferred_element_type=f32, precision=lax.Precision.HIGHEST) + b1d[...]
    q_out[...] = jnp.dot(zi2, w1i[...], preferred_element_type=f32, precision=lax.Precision.HIGHEST)


_fin = pl.pallas_call(
    _fin_body,
    grid=(GRID_TC,),
    in_specs=[
        pl.BlockSpec((NC, R_TC, W_AUG), lambda i: (0, i, 0)),
        pl.BlockSpec((NC, R_TC, W_AUG), lambda i: (0, i, 0)),
        pl.BlockSpec((1, H), lambda i: (0, 0)),
        pl.BlockSpec((1, H), lambda i: (0, 0)),
        pl.BlockSpec((H, H), lambda i: (0, 0)),
        pl.BlockSpec((H, H), lambda i: (0, 0)),
        pl.BlockSpec((1, H), lambda i: (0, 0)),
    ],
    out_specs=[pl.BlockSpec((R_TC, H), lambda i: (i, 0))] * 2,
    out_shape=[jax.ShapeDtypeStruct((N_NODES, H), jnp.float32)] * 2,
)


def _pad_edges(idx_row):
    return jnp.pad(idx_row, (0, E_PAD - N_EDGES))


def kernel(x_user, x_item, edge_index_rates, edge_index_rev,
           edge_label_index, params):
    c1r = params['c1_rates']
    c1v = params['c1_rev']
    c2r = params['c2_rates']
    c2v = params['c2_rev']
    edge_phase = _build_edge_phase()
    decoder = _build_decoder()

    src_r = _pad_edges(edge_index_rates[0])
    dst_r = _pad_edges(edge_index_rates[1])
    src_v = _pad_edges(edge_index_rev[0])
    dst_v = _pad_edges(edge_index_rev[1])
    row_d = _pad_edges(edge_label_index[0])
    col_d = _pad_edges(edge_label_index[1])

    hl1r, hr1r, hl1v, hr1v = _proj1(
        x_user, x_item,
        c1r['Wl'], c1r['bl'].reshape(1, H), c1r['Wr'], c1r['br'].reshape(1, H),
        c1v['Wl'], c1v['bl'].reshape(1, H), c1v['Wr'], c1v['br'].reshape(1, H))

    aug_r1 = edge_phase(hl1r, hr1r, src_r, dst_r, c1r['att'])
    aug_v1 = edge_phase(hl1v, hr1v, src_v, dst_v, c1v['att'])

    hl2r, hr2r, hl2v, hr2v = _mid(
        aug_r1, aug_v1, c1r['bias'].reshape(1, H), c1v['bias'].reshape(1, H),
        c2r['Wl'], c2r['bl'].reshape(1, H), c2r['Wr'], c2r['br'].reshape(1, H),
        c2v['Wl'], c2v['bl'].reshape(1, H), c2v['Wr'], c2v['br'].reshape(1, H))

    aug_r2 = edge_phase(hl2r, hr2r, src_r, dst_r, c2r['att'])
    aug_v2 = edge_phase(hl2v, hr2v, src_v, dst_v, c2v['att'])

    P, Q = _fin(aug_r2, aug_v2, c2r['bias'].reshape(1, H),
                c2v['bias'].reshape(1, H), params['dec_W1'][:H],
                params['dec_W1'][H:], params['dec_b1'].reshape(1, H))

    pred_pad = decoder(P, Q, row_d, col_d,
                       params['dec_W2'].reshape(H),
                       jnp.broadcast_to(params['dec_b2'], (L,)))
    pred = pred_pad[:N_EDGES]

    mask = jnp.ones((edge_label_index.shape[1],), dtype=bool)
    return (pred, mask)


# speedup vs baseline: 1.0036x; 1.0036x over previous
import functools

import jax
import jax.numpy as jnp
from jax import lax
from jax.experimental import pallas as pl
from jax.experimental.pallas import tpu as pltpu
from jax.experimental.pallas import tpu_sc as plsc

N_NODES = 10000
N_EDGES = 160000
H = 32
W_AUG = 48
NC, NS, L = 2, 16, 16
NW = NC * NS
EB = 128
EPT = 5120
E_PAD = NW * EPT
NBLK_FULL = EPT // EB
NBLK_LAST = (N_EDGES - (NW - 1) * EPT) // EB
N_PAD = 10240
ROWS_PER_SUB = N_PAD // NS
R_TC = 1000
GRID_TC = N_NODES // R_TC


def _sc_mesh():
    return plsc.VectorSubcoreMesh(
        core_axis_name="c", subcore_axis_name="s",
        num_cores=NC, num_subcores=NS)


def _vmem_to_vmem(dst, dst_off, src, src_off, n):
    for k in range(n // L):
        dst[pl.ds(dst_off + k * L, L)] = src[pl.ds(src_off + k * L, L)]


@functools.cache
def _build_edge_phase():
    @functools.partial(
        pl.kernel,
        out_type=jax.ShapeDtypeStruct((NC, N_PAD, W_AUG), jnp.float32),
        mesh=_sc_mesh(),
        compiler_params=pltpu.CompilerParams(use_tc_tiling_on_sc=False),
        scratch_types=[
            pltpu.VMEM((EPT,), jnp.int32),
            pltpu.VMEM((EPT,), jnp.int32),
            pltpu.VMEM((EB,), jnp.int32),
            pltpu.VMEM((EB,), jnp.int32),
            pltpu.VMEM((EB, H), jnp.float32),
            pltpu.VMEM((EB, H), jnp.float32),
            pltpu.VMEM((EB, H), jnp.float32),
            pltpu.VMEM((EB, H), jnp.float32),
            pltpu.VMEM((EB, W_AUG), jnp.float32),
            pltpu.VMEM((EB, W_AUG), jnp.float32),
            pltpu.VMEM((H,), jnp.float32),
            pltpu.VMEM((ROWS_PER_SUB, W_AUG), jnp.float32),
            pltpu.VMEM_SHARED((N_PAD, W_AUG), jnp.float32),
            pltpu.SemaphoreType.DMA,
            pltpu.SemaphoreType.DMA,
            pltpu.SemaphoreType.DMA,
            pltpu.SemaphoreType.DMA,
        ],
    )
    def _edge_phase(hl_hbm, hr_hbm, src_hbm, dst_hbm, att_hbm, aug_out,
                    sidx_all, didx_all, didx_a, didx_b,
                    hl_a, hl_b, hr_a, hr_b, pv_a, pv_b,
                    att_v, zero_v, acc_sh,
                    sem_ga, sem_gb, sem_sa, sem_sb):
        c = lax.axis_index("c")
        s = lax.axis_index("s")
        wid = s * NC + c
        base_e = wid * EPT
        nblk = jnp.where(wid == NW - 1, NBLK_LAST, NBLK_FULL)

        lane_i = lax.broadcasted_iota(jnp.int32, (L,), 0)
        zv = lane_i.astype(jnp.float32) * 0.0

        pltpu.sync_copy(src_hbm.at[pl.ds(base_e, EPT)], sidx_all)
        pltpu.sync_copy(dst_hbm.at[pl.ds(base_e, EPT)], didx_all)
        pltpu.sync_copy(att_hbm, att_v)
        att0 = att_v[pl.ds(0, L)]
        att1 = att_v[pl.ds(L, L)]
        onehot0 = jnp.where(lane_i == 0, 1.0, 0.0).astype(jnp.float32)

        def _zrow(i, carry):
            for k in range(W_AUG // L):
                zero_v[i, pl.ds(k * L, L)] = zv
            return carry

        lax.fori_loop(0, ROWS_PER_SUB, _zrow, 0)
        pltpu.sync_copy(zero_v,
                        acc_sh.at[pl.ds(s * ROWS_PER_SUB, ROWS_PER_SUB)])

        def _gissue(j, hl_p, hr_p, sem_g):
            pltpu.async_copy(
                hl_hbm.at[sidx_all.at[pl.ds(j * EB, EB)]], hl_p, sem_g)
            pltpu.async_copy(
                hr_hbm.at[didx_all.at[pl.ds(j * EB, EB)]], hr_p, sem_g)

        def _gwait(hl_p, hr_p, sem_g):
            pltpu.make_async_copy(hl_hbm.at[pl.ds(0, EB)], hl_p, sem_g).wait()
            pltpu.make_async_copy(hr_hbm.at[pl.ds(0, EB)], hr_p, sem_g).wait()

        def _compute(hl_p, hr_p, pv_p):
            @plsc.parallel_loop(0, EB, unroll=4)
            def _edge(e):
                hl0 = hl_p[e, pl.ds(0, L)]
                hl1 = hl_p[e, pl.ds(L, L)]
                hr0 = hr_p[e, pl.ds(0, L)]
                hr1 = hr_p[e, pl.ds(L, L)]
                s0 = hl0 + hr0
                s1 = hl1 + hr1
                t0 = jnp.maximum(s0, 0.2 * s0)
                t1 = jnp.maximum(s1, 0.2 * s1)
                u = t0 * att0 + t1 * att1
                for sh in (8, 4, 2, 1):
                    u = u + u.at[lane_i ^ sh].get(mode="promise_in_bounds")
                p = jnp.exp(u)
                pv_p[e, pl.ds(0, L)] = p * hl0
                pv_p[e, pl.ds(L, L)] = p * hl1
                pv_p[e, pl.ds(2 * L, L)] = p * onehot0

        def _swait(pv_p, didx_p, sem_s):
            pltpu.make_async_copy(pv_p, acc_sh.at[didx_p], sem_s).wait()

        plsc.subcore_barrier()

        _gissue(0, hl_a, hr_a, sem_ga)

        def _half(j, hl_p, hr_p, pv_p, didx_p, sem_g, sem_s,
                  hl_q, hr_q, sem_gq):
            @pl.when(j < nblk)
            def _():
                _gwait(hl_p, hr_p, sem_g)

                @pl.when(j + 1 < nblk)
                def _():
                    _gissue(j + 1, hl_q, hr_q, sem_gq)

                @pl.when(j >= 2)
                def _():
                    _swait(pv_p, didx_p, sem_s)

                _compute(hl_p, hr_p, pv_p)
                _vmem_to_vmem(didx_p, 0, didx_all, j * EB, EB)
                pltpu.async_copy(pv_p, acc_sh.at[didx_p], sem_s, add=True)

        def _pair(jj, carry):
            j0 = 2 * jj
            _half(j0, hl_a, hr_a, pv_a, didx_a, sem_ga, sem_sa,
                  hl_b, hr_b, sem_gb)
            _half(j0 + 1, hl_b, hr_b, pv_b, didx_b, sem_gb, sem_sb,
                  hl_a, hr_a, sem_ga)
            return carry

        lax.fori_loop(0, NBLK_FULL // 2, _pair, 0)

        _swait(pv_a, didx_a, sem_sa)
        _swait(pv_b, didx_b, sem_sb)

        plsc.subcore_barrier()
        pltpu.sync_copy(acc_sh.at[pl.ds(s * ROWS_PER_SUB, ROWS_PER_SUB)],
                        aug_out.at[c, pl.ds(s * ROWS_PER_SUB, ROWS_PER_SUB)])

    return _edge_phase


@functools.cache
def _build_decoder():
    @functools.partial(
        pl.kernel,
        out_type=jax.ShapeDtypeStruct((E_PAD,), jnp.float32),
        mesh=_sc_mesh(),
        compiler_params=pltpu.CompilerParams(use_tc_tiling_on_sc=False),
        scratch_types=[
            pltpu.VMEM((EPT,), jnp.int32),
            pltpu.VMEM((EPT,), jnp.int32),
            pltpu.VMEM((EB, H), jnp.float32),
            pltpu.VMEM((EB, H), jnp.float32),
            pltpu.VMEM((EB, H), jnp.float32),
            pltpu.VMEM((EB, H), jnp.float32),
            pltpu.VMEM((EPT,), jnp.float32),
            pltpu.VMEM((H,), jnp.float32),
            pltpu.VMEM((L,), jnp.float32),
            pltpu.SemaphoreType.DMA,
            pltpu.SemaphoreType.DMA,
        ],
    )
    def _decoder(p_hbm, q_hbm, row_hbm, col_hbm, w2_hbm, b2_hbm, pred_out,
                 ridx_all, cidx_all, pr_a, pr_b, qr_a, qr_b,
                 pred_v, w2_v, b2_v, sem_ga, sem_gb):
        c = lax.axis_index("c")
        s = lax.axis_index("s")
        wid = s * NC + c
        base_e = wid * EPT
        nblk = jnp.where(wid == NW - 1, NBLK_LAST, NBLK_FULL)

        pltpu.sync_copy(row_hbm.at[pl.ds(base_e, EPT)], ridx_all)
        pltpu.sync_copy(col_hbm.at[pl.ds(base_e, EPT)], cidx_all)
        pltpu.sync_copy(w2_hbm, w2_v)
        pltpu.sync_copy(b2_hbm, b2_v)
        w20 = w2_v[pl.ds(0, L)]
        w21 = w2_v[pl.ds(L, L)]
        b2 = b2_v[pl.ds(0, L)]
        lane_i = lax.broadcasted_iota(jnp.int32, (L,), 0)
        zv = lane_i.astype(jnp.float32) * 0.0

        def _gissue(j, pr_p, qr_p, sem_g):
            pltpu.async_copy(
                p_hbm.at[ridx_all.at[pl.ds(j * EB, EB)]], pr_p, sem_g)
            pltpu.async_copy(
                q_hbm.at[cidx_all.at[pl.ds(j * EB, EB)]], qr_p, sem_g)

        def _gwait(pr_p, qr_p, sem_g):
            pltpu.make_async_copy(p_hbm.at[pl.ds(0, EB)], pr_p, sem_g).wait()
            pltpu.make_async_copy(q_hbm.at[pl.ds(0, EB)], qr_p, sem_g).wait()

        def _compute(j, pr_p, qr_p):
            @plsc.parallel_loop(0, EB // L, unroll=4)
            def _grp(g):
                base = g * L
                pacc = zv
                for jj in range(L):
                    e = base + jj
                    p0 = pr_p[e, pl.ds(0, L)]
                    p1 = pr_p[e, pl.ds(L, L)]
                    q0 = qr_p[e, pl.ds(0, L)]
                    q1 = qr_p[e, pl.ds(L, L)]
                    t0 = jnp.maximum(p0 + q0, 0.0)
                    t1 = jnp.maximum(p1 + q1, 0.0)
                    u = t0 * w20 + t1 * w21
                    for sh in (8, 4, 2, 1):
                        u = u + u.at[lane_i ^ sh].get(
                            mode="promise_in_bounds")
                    pacc = jnp.where(lane_i == jj, u, pacc)
                pred_v[pl.ds(j * EB + base, L)] = pacc + b2

        _gissue(0, pr_a, qr_a, sem_ga)

        def _half(j, pr_p, qr_p, sem_g, pr_q, qr_q, sem_gq):
            @pl.when(j < nblk)
            def _():
                _gwait(pr_p, qr_p, sem_g)

                @pl.when(j + 1 < nblk)
                def _():
                    _gissue(j + 1, pr_q, qr_q, sem_gq)

                _compute(j, pr_p, qr_p)

        def _pair(jj, carry):
            j0 = 2 * jj
            _half(j0, pr_a, qr_a, sem_ga, pr_b, qr_b, sem_gb)
            _half(j0 + 1, pr_b, qr_b, sem_gb, pr_a, qr_a, sem_ga)
            return carry

        lax.fori_loop(0, NBLK_FULL // 2, _pair, 0)

        pltpu.sync_copy(pred_v, pred_out.at[pl.ds(base_e, EPT)])

    return _decoder


def _proj1_body(xu, xi, wlr, blr, wrr, brr, wlv, blv, wrv, brv,
                hlr, hrr, hlv, hrv):
    xu_ = xu[...]
    xi_ = xi[...]
    f32 = jnp.float32
    hlr[...] = jnp.dot(xu_, wlr[...], preferred_element_type=f32, precision=lax.Precision.HIGHEST) + blr[...]
    hrr[...] = jnp.dot(xi_, wrr[...], preferred_element_type=f32, precision=lax.Precision.HIGHEST) + brr[...]
    hlv[...] = jnp.dot(xi_, wlv[...], preferred_element_type=f32, precision=lax.Precision.HIGHEST) + blv[...]
    hrv[...] = jnp.dot(xu_, wrv[...], preferred_element_type=f32, precision=lax.Precision.HIGHEST) + brv[...]


_proj1 = pl.pallas_call(
    _proj1_body,
    grid=(GRID_TC,),
    in_specs=[
        pl.BlockSpec((R_TC, 128), lambda i: (i, 0)),
        pl.BlockSpec((R_TC, 256), lambda i: (i, 0)),
        pl.BlockSpec((128, H), lambda i: (0, 0)),
        pl.BlockSpec((1, H), lambda i: (0, 0)),
        pl.BlockSpec((256, H), lambda i: (0, 0)),
        pl.BlockSpec((1, H), lambda i: (0, 0)),
        pl.BlockSpec((256, H), lambda i: (0, 0)),
        pl.BlockSpec((1, H), lambda i: (0, 0)),
        pl.BlockSpec((128, H), lambda i: (0, 0)),
        pl.BlockSpec((1, H), lambda i: (0, 0)),
    ],
    out_specs=[pl.BlockSpec((R_TC, H), lambda i: (i, 0))] * 4,
    out_shape=[jax.ShapeDtypeStruct((N_NODES, H), jnp.float32)] * 4,
)


def _mid_body(augr, augv, b1r, b1v, wl2r, bl2r, wr2r, br2r,
              wl2v, bl2v, wr2v, br2v, hl2r, hr2r, hl2v, hr2v):
    f32 = jnp.float32
    ar = augr[0] + augr[1]
    av = augv[0] + augv[1]
    zi1 = jnp.maximum(ar[:, :H] / (ar[:, H:H + 1] + 1e-16) + b1r[...], 0.0)
    zu1 = jnp.maximum(av[:, :H] / (av[:, H:H + 1] + 1e-16) + b1v[...], 0.0)
    hl2r[...] = jnp.dot(zu1, wl2r[...], preferred_element_type=f32, precision=lax.Precision.HIGHEST) + bl2r[...]
    hr2r[...] = jnp.dot(zi1, wr2r[...], preferred_element_type=f32, precision=lax.Precision.HIGHEST) + br2r[...]
    hl2v[...] = jnp.dot(zi1, wl2v[...], preferred_element_type=f32, precision=lax.Precision.HIGHEST) + bl2v[...]
    hr2v[...] = jnp.dot(zu1, wr2v[...], preferred_element_type=f32, precision=lax.Precision.HIGHEST) + br2v[...]


_mid = pl.pallas_call(
    _mid_body,
    grid=(GRID_TC,),
    in_specs=[
        pl.BlockSpec((NC, R_TC, W_AUG), lambda i: (0, i, 0)),
        pl.BlockSpec((NC, R_TC, W_AUG), lambda i: (0, i, 0)),
        pl.BlockSpec((1, H), lambda i: (0, 0)),
        pl.BlockSpec((1, H), lambda i: (0, 0)),
    ] + [
        pl.BlockSpec((H, H), lambda i: (0, 0)),
        pl.BlockSpec((1, H), lambda i: (0, 0)),
    ] * 4,
    out_specs=[pl.BlockSpec((R_TC, H), lambda i: (i, 0))] * 4,
    out_shape=[jax.ShapeDtypeStruct((N_NODES, H), jnp.float32)] * 4,
)


def _fin_body(augr, augv, b2r, b2v, w1u, w1i, b1d, p_out, q_out):
    f32 = jnp.float32
    ar = augr[0] + augr[1]
    av = augv[0] + augv[1]
    zi2 = ar[:, :H] / (ar[:, H:H + 1] + 1e-16) + b2r[...]
    zu2 = av[:, :H] / (av[:, H:H + 1] + 1e-16) + b2v[...]
    p_out[...] = jnp.dot(zu2, w1u[...], preferred_element_type=f32, precision=lax.Precision.HIGHEST) + b1d[...]
    q_out[...] = jnp.dot(zi2, w1i[...], preferred_element_type=f32, precision=lax.Precision.HIGHEST)


_fin = pl.pallas_call(
    _fin_body,
    grid=(GRID_TC,),
    in_specs=[
        pl.BlockSpec((NC, R_TC, W_AUG), lambda i: (0, i, 0)),
        pl.BlockSpec((NC, R_TC, W_AUG), lambda i: (0, i, 0)),
        pl.BlockSpec((1, H), lambda i: (0, 0)),
        pl.BlockSpec((1, H), lambda i: (0, 0)),
        pl.BlockSpec((H, H), lambda i: (0, 0)),
        pl.BlockSpec((H, H), lambda i: (0, 0)),
        pl.BlockSpec((1, H), lambda i: (0, 0)),
    ],
    out_specs=[pl.BlockSpec((R_TC, H), lambda i: (i, 0))] * 2,
    out_shape=[jax.ShapeDtypeStruct((N_NODES, H), jnp.float32)] * 2,
)


def _pad_edges(idx_row):
    return jnp.pad(idx_row, (0, E_PAD - N_EDGES))


def kernel(x_user, x_item, edge_index_rates, edge_index_rev,
           edge_label_index, params):
    c1r = params['c1_rates']
    c1v = params['c1_rev']
    c2r = params['c2_rates']
    c2v = params['c2_rev']
    edge_phase = _build_edge_phase()
    decoder = _build_decoder()

    src_r = _pad_edges(edge_index_rates[0])
    dst_r = _pad_edges(edge_index_rates[1])
    src_v = _pad_edges(edge_index_rev[0])
    dst_v = _pad_edges(edge_index_rev[1])
    row_d = _pad_edges(edge_label_index[0])
    col_d = _pad_edges(edge_label_index[1])

    hl1r, hr1r, hl1v, hr1v = _proj1(
        x_user, x_item,
        c1r['Wl'], c1r['bl'].reshape(1, H), c1r['Wr'], c1r['br'].reshape(1, H),
        c1v['Wl'], c1v['bl'].reshape(1, H), c1v['Wr'], c1v['br'].reshape(1, H))

    aug_r1 = edge_phase(hl1r, hr1r, src_r, dst_r, c1r['att'])
    aug_v1 = edge_phase(hl1v, hr1v, src_v, dst_v, c1v['att'])

    hl2r, hr2r, hl2v, hr2v = _mid(
        aug_r1, aug_v1, c1r['bias'].reshape(1, H), c1v['bias'].reshape(1, H),
        c2r['Wl'], c2r['bl'].reshape(1, H), c2r['Wr'], c2r['br'].reshape(1, H),
        c2v['Wl'], c2v['bl'].reshape(1, H), c2v['Wr'], c2v['br'].reshape(1, H))

    aug_r2 = edge_phase(hl2r, hr2r, src_r, dst_r, c2r['att'])
    aug_v2 = edge_phase(hl2v, hr2v, src_v, dst_v, c2v['att'])

    P, Q = _fin(aug_r2, aug_v2, c2r['bias'].reshape(1, H),
                c2v['bias'].reshape(1, H), params['dec_W1'][:H],
                params['dec_W1'][H:], params['dec_b1'].reshape(1, H))

    pred_pad = decoder(P, Q, row_d, col_d,
                       params['dec_W2'].reshape(H),
                       jnp.broadcast_to(params['dec_b2'], (L,)))
    pred = pred_pad[:N_EDGES]

    mask = jnp.ones((edge_label_index.shape[1],), dtype=bool)
    return (pred, mask)


# speedup vs baseline: 1.1306x; 1.1266x over previous
import functools

import jax
import jax.numpy as jnp
from jax import lax
from jax.experimental import pallas as pl
from jax.experimental.pallas import tpu as pltpu
from jax.experimental.pallas import tpu_sc as plsc

N_NODES = 10000
N_EDGES = 160000
H = 32
W_AUG = 48
NC, NS, L = 2, 16, 16
NW = NC * NS
EB = 128
EPT = 5120
E_PAD = NW * EPT
NBLK_FULL = EPT // EB
NBLK_LAST = (N_EDGES - (NW - 1) * EPT) // EB
N_PAD = 10240
ROWS_PER_SUB = N_PAD // NS
R_TC = 1000
GRID_TC = N_NODES // R_TC


def _sc_mesh():
    return plsc.VectorSubcoreMesh(
        core_axis_name="c", subcore_axis_name="s",
        num_cores=NC, num_subcores=NS)


def _vmem_to_vmem(dst, dst_off, src, src_off, n):
    for k in range(n // L):
        dst[pl.ds(dst_off + k * L, L)] = src[pl.ds(src_off + k * L, L)]


@functools.cache
def _build_edge_phase():
    @functools.partial(
        pl.kernel,
        out_type=jax.ShapeDtypeStruct((NC, N_PAD, W_AUG), jnp.float32),
        mesh=_sc_mesh(),
        compiler_params=pltpu.CompilerParams(use_tc_tiling_on_sc=False),
        scratch_types=[
            pltpu.VMEM((EPT,), jnp.int32),
            pltpu.VMEM((EPT,), jnp.int32),
            pltpu.VMEM((EB,), jnp.int32),
            pltpu.VMEM((EB,), jnp.int32),
            pltpu.VMEM((EB, H), jnp.float32),
            pltpu.VMEM((EB, H), jnp.float32),
            pltpu.VMEM((EB, H), jnp.float32),
            pltpu.VMEM((EB, H), jnp.float32),
            pltpu.VMEM((EB, W_AUG), jnp.float32),
            pltpu.VMEM((EB, W_AUG), jnp.float32),
            pltpu.VMEM((H,), jnp.float32),
            pltpu.VMEM((ROWS_PER_SUB, W_AUG), jnp.float32),
            pltpu.VMEM_SHARED((N_PAD, W_AUG), jnp.float32),
            pltpu.SemaphoreType.DMA,
            pltpu.SemaphoreType.DMA,
            pltpu.SemaphoreType.DMA,
            pltpu.SemaphoreType.DMA,
        ],
    )
    def _edge_phase(hl_hbm, hr_hbm, src_hbm, dst_hbm, att_hbm, aug_out,
                    sidx_all, didx_all, didx_a, didx_b,
                    hl_a, hl_b, hr_a, hr_b, pv_a, pv_b,
                    att_v, zero_v, acc_sh,
                    sem_ga, sem_gb, sem_sa, sem_sb):
        c = lax.axis_index("c")
        s = lax.axis_index("s")
        wid = s * NC + c
        base_e = wid * EPT
        nblk = jnp.where(wid == NW - 1, NBLK_LAST, NBLK_FULL)

        lane_i = lax.broadcasted_iota(jnp.int32, (L,), 0)
        zv = lane_i.astype(jnp.float32) * 0.0

        pltpu.sync_copy(src_hbm.at[pl.ds(base_e, EPT)], sidx_all)
        pltpu.sync_copy(dst_hbm.at[pl.ds(base_e, EPT)], didx_all)
        pltpu.sync_copy(att_hbm, att_v)
        att0 = att_v[pl.ds(0, L)]
        att1 = att_v[pl.ds(L, L)]
        onehot0 = jnp.where(lane_i == 0, 1.0, 0.0).astype(jnp.float32)

        def _zrow(i, carry):
            for k in range(W_AUG // L):
                zero_v[i, pl.ds(k * L, L)] = zv
            return carry

        lax.fori_loop(0, ROWS_PER_SUB, _zrow, 0)
        pltpu.sync_copy(zero_v,
                        acc_sh.at[pl.ds(s * ROWS_PER_SUB, ROWS_PER_SUB)])

        def _gissue(j, hl_p, hr_p, sem_g):
            pltpu.async_copy(
                hl_hbm.at[sidx_all.at[pl.ds(j * EB, EB)]], hl_p, sem_g)
            pltpu.async_copy(
                hr_hbm.at[didx_all.at[pl.ds(j * EB, EB)]], hr_p, sem_g)

        def _gwait(hl_p, hr_p, sem_g):
            pltpu.make_async_copy(hl_hbm.at[pl.ds(0, EB)], hl_p, sem_g).wait()
            pltpu.make_async_copy(hr_hbm.at[pl.ds(0, EB)], hr_p, sem_g).wait()

        def _compute(hl_p, hr_p, pv_p):
            @plsc.parallel_loop(0, EB, unroll=4)
            def _edge(e):
                hl0 = hl_p[e, pl.ds(0, L)]
                hl1 = hl_p[e, pl.ds(L, L)]
                hr0 = hr_p[e, pl.ds(0, L)]
                hr1 = hr_p[e, pl.ds(L, L)]
                s0 = hl0 + hr0
                s1 = hl1 + hr1
                t0 = jnp.maximum(s0, 0.2 * s0)
                t1 = jnp.maximum(s1, 0.2 * s1)
                u = t0 * att0 + t1 * att1
                for sh in (8, 4, 2, 1):
                    u = u + u.at[lane_i ^ sh].get(mode="promise_in_bounds")
                p = jnp.exp(u)
                pv_p[e, pl.ds(0, L)] = p * hl0
                pv_p[e, pl.ds(L, L)] = p * hl1
                pv_p[e, pl.ds(2 * L, L)] = p * onehot0

        def _swait(pv_p, didx_p, sem_s):
            pltpu.make_async_copy(pv_p, acc_sh.at[didx_p], sem_s).wait()

        plsc.subcore_barrier()

        _gissue(0, hl_a, hr_a, sem_ga)

        def _half(j, hl_p, hr_p, pv_p, didx_p, sem_g, sem_s,
                  hl_q, hr_q, sem_gq):
            @pl.when(j < nblk)
            def _():
                _gwait(hl_p, hr_p, sem_g)

                @pl.when(j + 1 < nblk)
                def _():
                    _gissue(j + 1, hl_q, hr_q, sem_gq)

                @pl.when(j >= 2)
                def _():
                    _swait(pv_p, didx_p, sem_s)

                _compute(hl_p, hr_p, pv_p)
                _vmem_to_vmem(didx_p, 0, didx_all, j * EB, EB)
                pltpu.async_copy(pv_p, acc_sh.at[didx_p], sem_s, add=True)

        def _pair(jj, carry):
            j0 = 2 * jj
            _half(j0, hl_a, hr_a, pv_a, didx_a, sem_ga, sem_sa,
                  hl_b, hr_b, sem_gb)
            _half(j0 + 1, hl_b, hr_b, pv_b, didx_b, sem_gb, sem_sb,
                  hl_a, hr_a, sem_ga)
            return carry

        lax.fori_loop(0, NBLK_FULL // 2, _pair, 0)

        _swait(pv_a, didx_a, sem_sa)
        _swait(pv_b, didx_b, sem_sb)

        plsc.subcore_barrier()
        pltpu.sync_copy(acc_sh.at[pl.ds(s * ROWS_PER_SUB, ROWS_PER_SUB)],
                        aug_out.at[c, pl.ds(s * ROWS_PER_SUB, ROWS_PER_SUB)])

    return _edge_phase


@functools.cache
def _build_decoder():
    @functools.partial(
        pl.kernel,
        out_type=jax.ShapeDtypeStruct((E_PAD,), jnp.float32),
        mesh=_sc_mesh(),
        compiler_params=pltpu.CompilerParams(use_tc_tiling_on_sc=False),
        scratch_types=[
            pltpu.VMEM((EPT,), jnp.int32),
            pltpu.VMEM((EPT,), jnp.int32),
            pltpu.VMEM((EB, H), jnp.float32),
            pltpu.VMEM((EB, H), jnp.float32),
            pltpu.VMEM((EB, H), jnp.float32),
            pltpu.VMEM((EB, H), jnp.float32),
            pltpu.VMEM((EPT,), jnp.float32),
            pltpu.VMEM((H,), jnp.float32),
            pltpu.VMEM((L,), jnp.float32),
            pltpu.SemaphoreType.DMA,
            pltpu.SemaphoreType.DMA,
        ],
    )
    def _decoder(p_hbm, q_hbm, row_hbm, col_hbm, w2_hbm, b2_hbm, pred_out,
                 ridx_all, cidx_all, pr_a, pr_b, qr_a, qr_b,
                 pred_v, w2_v, b2_v, sem_ga, sem_gb):
        c = lax.axis_index("c")
        s = lax.axis_index("s")
        wid = s * NC + c
        base_e = wid * EPT
        nblk = jnp.where(wid == NW - 1, NBLK_LAST, NBLK_FULL)

        pltpu.sync_copy(row_hbm.at[pl.ds(base_e, EPT)], ridx_all)
        pltpu.sync_copy(col_hbm.at[pl.ds(base_e, EPT)], cidx_all)
        pltpu.sync_copy(w2_hbm, w2_v)
        pltpu.sync_copy(b2_hbm, b2_v)
        w20 = w2_v[pl.ds(0, L)]
        w21 = w2_v[pl.ds(L, L)]
        b2 = b2_v[pl.ds(0, L)]
        lane_i = lax.broadcasted_iota(jnp.int32, (L,), 0)
        zv = lane_i.astype(jnp.float32) * 0.0

        def _gissue(j, pr_p, qr_p, sem_g):
            pltpu.async_copy(
                p_hbm.at[ridx_all.at[pl.ds(j * EB, EB)]], pr_p, sem_g)
            pltpu.async_copy(
                q_hbm.at[cidx_all.at[pl.ds(j * EB, EB)]], qr_p, sem_g)

        def _gwait(pr_p, qr_p, sem_g):
            pltpu.make_async_copy(p_hbm.at[pl.ds(0, EB)], pr_p, sem_g).wait()
            pltpu.make_async_copy(q_hbm.at[pl.ds(0, EB)], qr_p, sem_g).wait()

        def _compute(j, pr_p, qr_p):
            @plsc.parallel_loop(0, EB // L, unroll=4)
            def _grp(g):
                base = g * L
                pacc = zv
                for jj in range(L):
                    e = base + jj
                    p0 = pr_p[e, pl.ds(0, L)]
                    p1 = pr_p[e, pl.ds(L, L)]
                    q0 = qr_p[e, pl.ds(0, L)]
                    q1 = qr_p[e, pl.ds(L, L)]
                    t0 = jnp.maximum(p0 + q0, 0.0)
                    t1 = jnp.maximum(p1 + q1, 0.0)
                    u = t0 * w20 + t1 * w21
                    for sh in (8, 4, 2, 1):
                        u = u + u.at[lane_i ^ sh].get(
                            mode="promise_in_bounds")
                    pacc = jnp.where(lane_i == jj, u, pacc)
                pred_v[pl.ds(j * EB + base, L)] = pacc + b2

        _gissue(0, pr_a, qr_a, sem_ga)

        def _half(j, pr_p, qr_p, sem_g, pr_q, qr_q, sem_gq):
            @pl.when(j < nblk)
            def _():
                _gwait(pr_p, qr_p, sem_g)

                @pl.when(j + 1 < nblk)
                def _():
                    _gissue(j + 1, pr_q, qr_q, sem_gq)

                _compute(j, pr_p, qr_p)

        def _pair(jj, carry):
            j0 = 2 * jj
            _half(j0, pr_a, qr_a, sem_ga, pr_b, qr_b, sem_gb)
            _half(j0 + 1, pr_b, qr_b, sem_gb, pr_a, qr_a, sem_ga)
            return carry

        lax.fori_loop(0, NBLK_FULL // 2, _pair, 0)

        pltpu.sync_copy(pred_v, pred_out.at[pl.ds(base_e, EPT)])

    return _decoder


def _proj1_body(xu, xi, wlr, blr, wrr, brr, wlv, blv, wrv, brv,
                hlr, hrr, hlv, hrv):
    xu_ = xu[...]
    xi_ = xi[...]
    f32 = jnp.float32
    hlr[...] = jnp.dot(xu_, wlr[...], preferred_element_type=f32) + blr[...]
    hrr[...] = jnp.dot(xi_, wrr[...], preferred_element_type=f32) + brr[...]
    hlv[...] = jnp.dot(xi_, wlv[...], preferred_element_type=f32) + blv[...]
    hrv[...] = jnp.dot(xu_, wrv[...], preferred_element_type=f32) + brv[...]


_proj1 = pl.pallas_call(
    _proj1_body,
    grid=(GRID_TC,),
    in_specs=[
        pl.BlockSpec((R_TC, 128), lambda i: (i, 0)),
        pl.BlockSpec((R_TC, 256), lambda i: (i, 0)),
        pl.BlockSpec((128, H), lambda i: (0, 0)),
        pl.BlockSpec((1, H), lambda i: (0, 0)),
        pl.BlockSpec((256, H), lambda i: (0, 0)),
        pl.BlockSpec((1, H), lambda i: (0, 0)),
        pl.BlockSpec((256, H), lambda i: (0, 0)),
        pl.BlockSpec((1, H), lambda i: (0, 0)),
        pl.BlockSpec((128, H), lambda i: (0, 0)),
        pl.BlockSpec((1, H), lambda i: (0, 0)),
    ],
    out_specs=[pl.BlockSpec((R_TC, H), lambda i: (i, 0))] * 4,
    out_shape=[jax.ShapeDtypeStruct((N_NODES, H), jnp.float32)] * 4,
)


def _mid_body(augr, augv, b1r, b1v, wl2r, bl2r, wr2r, br2r,
              wl2v, bl2v, wr2v, br2v, hl2r, hr2r, hl2v, hr2v):
    f32 = jnp.float32
    ar = augr[0] + augr[1]
    av = augv[0] + augv[1]
    zi1 = jnp.maximum(ar[:, :H] / (ar[:, H:H + 1] + 1e-30) + b1r[...], 0.0)
    zu1 = jnp.maximum(av[:, :H] / (av[:, H:H + 1] + 1e-30) + b1v[...], 0.0)
    hl2r[...] = jnp.dot(zu1, wl2r[...], preferred_element_type=f32) + bl2r[...]
    hr2r[...] = jnp.dot(zi1, wr2r[...], preferred_element_type=f32) + br2r[...]
    hl2v[...] = jnp.dot(zi1, wl2v[...], preferred_element_type=f32) + bl2v[...]
    hr2v[...] = jnp.dot(zu1, wr2v[...], preferred_element_type=f32) + br2v[...]


_mid = pl.pallas_call(
    _mid_body,
    grid=(GRID_TC,),
    in_specs=[
        pl.BlockSpec((NC, R_TC, W_AUG), lambda i: (0, i, 0)),
        pl.BlockSpec((NC, R_TC, W_AUG), lambda i: (0, i, 0)),
        pl.BlockSpec((1, H), lambda i: (0, 0)),
        pl.BlockSpec((1, H), lambda i: (0, 0)),
    ] + [
        pl.BlockSpec((H, H), lambda i: (0, 0)),
        pl.BlockSpec((1, H), lambda i: (0, 0)),
    ] * 4,
    out_specs=[pl.BlockSpec((R_TC, H), lambda i: (i, 0))] * 4,
    out_shape=[jax.ShapeDtypeStruct((N_NODES, H), jnp.float32)] * 4,
)


def _fin_body(augr, augv, b2r, b2v, w1u, w1i, b1d, p_out, q_out):
    f32 = jnp.float32
    ar = augr[0] + augr[1]
    av = augv[0] + augv[1]
    zi2 = ar[:, :H] / (ar[:, H:H + 1] + 1e-30) + b2r[...]
    zu2 = av[:, :H] / (av[:, H:H + 1] + 1e-30) + b2v[...]
    p_out[...] = jnp.dot(zu2, w1u[...], preferred_element_type=f32) + b1d[...]
    q_out[...] = jnp.dot(zi2, w1i[...], preferred_element_type=f32)


_fin = pl.pallas_call(
    _fin_body,
    grid=(GRID_TC,),
    in_specs=[
        pl.BlockSpec((NC, R_TC, W_AUG), lambda i: (0, i, 0)),
        pl.BlockSpec((NC, R_TC, W_AUG), lambda i: (0, i, 0)),
        pl.BlockSpec((1, H), lambda i: (0, 0)),
        pl.BlockSpec((1, H), lambda i: (0, 0)),
        pl.BlockSpec((H, H), lambda i: (0, 0)),
        pl.BlockSpec((H, H), lambda i: (0, 0)),
        pl.BlockSpec((1, H), lambda i: (0, 0)),
    ],
    out_specs=[pl.BlockSpec((R_TC, H), lambda i: (i, 0))] * 2,
    out_shape=[jax.ShapeDtypeStruct((N_NODES, H), jnp.float32)] * 2,
)


def _pad_edges(idx_row):
    return jnp.pad(idx_row, (0, E_PAD - N_EDGES))


def kernel(x_user, x_item, edge_index_rates, edge_index_rev,
           edge_label_index, params):
    c1r = params['c1_rates']
    c1v = params['c1_rev']
    c2r = params['c2_rates']
    c2v = params['c2_rev']
    edge_phase = _build_edge_phase()
    decoder = _build_decoder()

    src_r = _pad_edges(edge_index_rates[0])
    dst_r = _pad_edges(edge_index_rates[1])
    src_v = _pad_edges(edge_index_rev[0])
    dst_v = _pad_edges(edge_index_rev[1])
    row_d = _pad_edges(edge_label_index[0])
    col_d = _pad_edges(edge_label_index[1])

    hl1r, hr1r, hl1v, hr1v = _proj1(
        x_user, x_item,
        c1r['Wl'], c1r['bl'].reshape(1, H), c1r['Wr'], c1r['br'].reshape(1, H),
        c1v['Wl'], c1v['bl'].reshape(1, H), c1v['Wr'], c1v['br'].reshape(1, H))

    aug_r1 = edge_phase(hl1r, hr1r, src_r, dst_r, c1r['att'])
    aug_v1 = edge_phase(hl1v, hr1v, src_v, dst_v, c1v['att'])

    hl2r, hr2r, hl2v, hr2v = _mid(
        aug_r1, aug_v1, c1r['bias'].reshape(1, H), c1v['bias'].reshape(1, H),
        c2r['Wl'], c2r['bl'].reshape(1, H), c2r['Wr'], c2r['br'].reshape(1, H),
        c2v['Wl'], c2v['bl'].reshape(1, H), c2v['Wr'], c2v['br'].reshape(1, H))

    aug_r2 = edge_phase(hl2r, hr2r, src_r, dst_r, c2r['att'])
    aug_v2 = edge_phase(hl2v, hr2v, src_v, dst_v, c2v['att'])

    P, Q = _fin(aug_r2, aug_v2, c2r['bias'].reshape(1, H),
                c2v['bias'].reshape(1, H), params['dec_W1'][:H],
                params['dec_W1'][H:], params['dec_b1'].reshape(1, H))

    pred_pad = decoder(P, Q, row_d, col_d,
                       params['dec_W2'].reshape(H),
                       jnp.broadcast_to(params['dec_b2'], (L,)))
    pred = pred_pad[:N_EDGES]

    mask = jnp.ones((edge_label_index.shape[1],), dtype=bool)
    return (pred, mask)
